# baseline (speedup 1.0000x reference)
import jax
import jax.numpy as jnp
from jax.experimental import pallas as pl

_B, _C, _F, _N = 4, 128, 12, 512
_K = 4
_R = _N // _K


def _body(*refs):
    y_refs = refs[:_K]
    x_ref, we_ref, wl_ref, b_ref, o_ref = refs[_K:]
    b = pl.program_id(0)
    f = pl.program_id(1)
    x = x_ref[b, :, f, :]
    m = jnp.zeros((_C, _N), jnp.float32)
    for k in range(_K):
        a = y_refs[k][0, 0] * we_ref[pl.ds(k * _R, _R), :]
        m = m + jnp.dot(x[:, k * _R:(k + 1) * _R], a,
                        preferred_element_type=jnp.float32)
    m = m * jnp.float32(1.0 / _N)
    h = jnp.dot(wl_ref[...], m,
                preferred_element_type=jnp.float32) + b_ref[...]
    o_ref[b, :, f, :] = jnp.maximum(h, 0.0)


@jax.jit
def kernel(Y, infos, W_edge, W_lin, b_lin):
    b2 = b_lin.reshape(_C, 1)
    grid = (_B, _F)
    y_specs = [
        pl.BlockSpec((1, 1, _R, _N), lambda b, f, k=k: (b, f, k, 0))
        for k in range(_K)
    ]
    return pl.pallas_call(
        _body,
        grid=grid,
        in_specs=y_specs + [
            pl.BlockSpec((_B, _C, _F, _N), lambda b, f: (0, 0, 0, 0)),
            pl.BlockSpec((_N, _N), lambda b, f: (0, 0)),
            pl.BlockSpec((_C, _C), lambda b, f: (0, 0)),
            pl.BlockSpec((_C, 1), lambda b, f: (0, 0)),
        ],
        out_specs=pl.BlockSpec((_B, _C, _F, _N), lambda b, f: (0, 0, 0, 0)),
        out_shape=jax.ShapeDtypeStruct((_B, _C, _F, _N), jnp.float32),
    )(*([Y] * _K), infos, W_edge, W_lin, b2)


# speedup vs baseline: 1.2888x; 1.2888x over previous
import jax
import jax.numpy as jnp
from jax.experimental import pallas as pl

_B, _C, _F, _N = 4, 128, 12, 512
_K = 4
_R = _N // _K


def _body(*refs):
    y_refs = refs[:_K]
    x_ref, we_ref, wl_ref, b_ref, o_ref = refs[_K:]
    b = pl.program_id(0)
    f = pl.program_id(1)
    x = x_ref[0, 0]
    m = jnp.zeros((_C, _N), jnp.float32)
    for k in range(_K):
        a = y_refs[k][0, 0] * we_ref[pl.ds(k * _R, _R), :]
        m = m + jnp.dot(x[:, k * _R:(k + 1) * _R], a,
                        preferred_element_type=jnp.float32)
    m = m * jnp.float32(1.0 / _N)
    h = jnp.dot(wl_ref[...], m,
                preferred_element_type=jnp.float32) + b_ref[...]
    o_ref[b, :, f, :] = jnp.maximum(h, 0.0)


@jax.jit
def kernel(Y, infos, W_edge, W_lin, b_lin):
    b2 = b_lin.reshape(_C, 1)
    grid = (_B, _F)
    y_specs = [
        pl.BlockSpec((1, 1, _R, _N), lambda b, f, k=k: (b, f, k, 0))
        for k in range(_K)
    ]
    return pl.pallas_call(
        _body,
        grid=grid,
        in_specs=y_specs + [
            pl.BlockSpec((1, 1, _C, _N), lambda b, f: (b, f, 0, 0)),
            pl.BlockSpec((_N, _N), lambda b, f: (0, 0)),
            pl.BlockSpec((_C, _C), lambda b, f: (0, 0)),
            pl.BlockSpec((_C, 1), lambda b, f: (0, 0)),
        ],
        out_specs=pl.BlockSpec((_B, _C, _F, _N), lambda b, f: (0, 0, 0, 0)),
        out_shape=jax.ShapeDtypeStruct((_B, _C, _F, _N), jnp.float32),
    )(*([Y] * _K), jnp.transpose(infos, (0, 2, 1, 3)), W_edge, W_lin, b2)


# speedup vs baseline: 1.7743x; 1.3767x over previous
import jax
import jax.numpy as jnp
from jax.experimental import pallas as pl

_B, _C, _F, _N = 4, 128, 12, 512
_K = 4
_R = _N // _K


def _body(*refs):
    y_refs = refs[:_K]
    x_ref, we_ref, wl_ref, b_ref, o_ref = refs[_K:]
    x = x_ref[0, 0]
    m = jnp.zeros((_C, _N), jnp.float32)
    for k in range(_K):
        a = y_refs[k][0, 0] * we_ref[pl.ds(k * _R, _R), :]
        m = m + jnp.dot(x[:, k * _R:(k + 1) * _R], a,
                        preferred_element_type=jnp.float32)
    m = m * jnp.float32(1.0 / _N)
    h = jnp.dot(wl_ref[...], m,
                preferred_element_type=jnp.float32) + b_ref[...]
    o_ref[0, 0] = jnp.maximum(h, 0.0)


@jax.jit
def kernel(Y, infos, W_edge, W_lin, b_lin):
    b2 = b_lin.reshape(_C, 1)
    grid = (_B, _F)
    y_specs = [
        pl.BlockSpec((1, 1, _R, _N), lambda b, f, k=k: (b, f, k, 0))
        for k in range(_K)
    ]
    out = pl.pallas_call(
        _body,
        grid=grid,
        in_specs=y_specs + [
            pl.BlockSpec((1, 1, _C, _N), lambda b, f: (b, f, 0, 0)),
            pl.BlockSpec((_N, _N), lambda b, f: (0, 0)),
            pl.BlockSpec((_C, _C), lambda b, f: (0, 0)),
            pl.BlockSpec((_C, 1), lambda b, f: (0, 0)),
        ],
        out_specs=pl.BlockSpec((1, 1, _C, _N), lambda b, f: (b, f, 0, 0)),
        out_shape=jax.ShapeDtypeStruct((_B, _F, _C, _N), jnp.float32),
    )(*([Y] * _K), jnp.transpose(infos, (0, 2, 1, 3)), W_edge, W_lin, b2)
    return jnp.transpose(out, (0, 2, 1, 3))
